# baseline (speedup 1.0000x reference)
import functools

import jax
import jax.numpy as jnp
from jax import lax
from jax.experimental import pallas as pl
from jax.experimental.pallas import tpu as pltpu
from jax.experimental.pallas import tpu_sc as plsc

_C, _T, _H, _W = 3, 64, 256, 256
_TS = _T // 4
_IDX = (0, 4, 8, 12, 16, 21, 25, 29, 33, 37, 42, 46, 50, 54, 58, 63)

_NSLOW = _C * _TS
_NC, _NS = 2, 16
_NW = _NC * _NS
_HC = 128
_CPR = _H // _HC
_NBUF = 3


def _gather_body(x, slow, bufs, sems_in, sems_out):
    cid = lax.axis_index("c")
    sid = lax.axis_index("s")
    w = sid * _NC + cid
    for j0 in range(_NSLOW):
        owner = j0 % _NW
        rows = [j0] if j0 >= _NW else None
        if j0 >= _NW:
            continue
        rows = [j0] + ([j0 + _NW] if j0 + _NW < _NSLOW else [])

        @pl.when(w == owner)
        def _(rows=rows):
            chunks = []
            for j in rows:
                c = j // _TS
                t_src = _IDX[j % _TS]
                t_dst = j % _TS
                for k in range(_CPR):
                    h0 = k * _HC
                    chunks.append((
                        x.at[c, pl.ds(t_src, 1), pl.ds(h0, _HC)],
                        slow.at[c, pl.ds(t_dst, 1), pl.ds(h0, _HC)],
                    ))
            n = len(chunks)
            ins = []
            outs = []
            for k, (src, dst) in enumerate(chunks):
                b = k % _NBUF
                ins.append(pltpu.make_async_copy(src, bufs[b], sems_in[b]))
                outs.append(pltpu.make_async_copy(bufs[b], dst, sems_out[b]))
            ins[0].start()
            if n > 1:
                ins[1].start()
            for k in range(n):
                ins[k].wait()
                outs[k].start()
                if k + 2 < n:
                    if k + 2 >= _NBUF:
                        outs[k + 2 - _NBUF].wait()
                    ins[k + 2].start()
            for k in range(max(0, n - _NBUF), n):
                outs[k].wait()


@functools.partial(
    pl.kernel,
    out_type=jax.ShapeDtypeStruct((_C, _TS, _H, _W), jnp.float32),
    mesh=plsc.VectorSubcoreMesh(core_axis_name="c", subcore_axis_name="s"),
    scratch_types=[
        pltpu.VMEM((1, _HC, _W), jnp.float32),
        pltpu.VMEM((1, _HC, _W), jnp.float32),
        pltpu.VMEM((1, _HC, _W), jnp.float32),
        pltpu.SemaphoreType.DMA,
        pltpu.SemaphoreType.DMA,
        pltpu.SemaphoreType.DMA,
        pltpu.SemaphoreType.DMA,
        pltpu.SemaphoreType.DMA,
        pltpu.SemaphoreType.DMA,
    ],
)
def _gather_sc(x, slow, b0, b1, b2, si0, si1, si2, so0, so1, so2):
    _gather_body(x, slow, (b0, b1, b2), (si0, si1, si2), (so0, so1, so2))


def _copy_tc_body(x_ref, out_ref):
    out_ref[...] = x_ref[...]


_TBLK = 4


def _copy_tc(x):
    return pl.pallas_call(
        _copy_tc_body,
        grid=(_C, _T // _TBLK),
        in_specs=[pl.BlockSpec((1, _TBLK, _H, _W), lambda c, i: (c, i, 0, 0))],
        out_specs=pl.BlockSpec((1, _TBLK, _H, _W), lambda c, i: (c, i, 0, 0)),
        out_shape=jax.ShapeDtypeStruct((_C, _T, _H, _W), jnp.float32),
    )(x)


def kernel(frames):
    slow = _gather_sc(frames)
    fast = _copy_tc(frames)
    return (slow, fast)


# speedup vs baseline: 1.1273x; 1.1273x over previous
import functools

import jax
import jax.numpy as jnp
from jax import lax
from jax.experimental import pallas as pl
from jax.experimental.pallas import tpu as pltpu
from jax.experimental.pallas import tpu_sc as plsc

_C, _T, _H, _W = 3, 64, 256, 256
_TS = _T // 4

_NROWS = _C * _T
_NC, _NS = 2, 16
_NW = _NC * _NS
_FPW = _NROWS // _NW
_HC = 64
_CPR = _H // _HC
_NBUF = 6
_RA = _NBUF - 2
_NCHUNK = _FPW * _CPR


def _body(x, slow, fast, bufs, sems_in, sems_out):
    cid = lax.axis_index("c")
    sid = lax.axis_index("s")
    w = sid * _NC + cid

    ins = []
    out_fast = []
    out_slow = []
    slow_flags = []
    for k in range(_NCHUNK):
        r = w * _FPW + (k // _CPR)
        c = r // _T
        t = r % _T
        h0 = (k % _CPR) * _HC
        b = k % _NBUF
        src = x.at[c, pl.ds(t, 1), pl.ds(h0, _HC)]
        ins.append(pltpu.make_async_copy(src, bufs[b], sems_in[b]))
        out_fast.append(pltpu.make_async_copy(
            bufs[b], fast.at[c, pl.ds(t, 1), pl.ds(h0, _HC)], sems_out[b]))
        kk = (5 * t + 10) // 21
        slow_flags.append((21 * kk) // 5 == t)
        out_slow.append(pltpu.make_async_copy(
            bufs[b], slow.at[c, pl.ds(kk, 1), pl.ds(h0, _HC)], sems_out[b]))

    def drain(j):
        out_fast[j].wait()

        @pl.when(slow_flags[j])
        def _():
            out_slow[j].wait()

    for k in range(min(_RA, _NCHUNK)):
        ins[k].start()
    for k in range(_NCHUNK):
        ins[k].wait()
        out_fast[k].start()

        @pl.when(slow_flags[k])
        def _(k=k):
            out_slow[k].start()

        nxt = k + _RA
        if nxt < _NCHUNK:
            if nxt - _NBUF >= 0:
                drain(nxt - _NBUF)
            ins[nxt].start()
    for j in range(max(0, _NCHUNK - _NBUF), _NCHUNK):
        drain(j)


@functools.partial(
    pl.kernel,
    out_type=(
        jax.ShapeDtypeStruct((_C, _TS, _H, _W), jnp.float32),
        jax.ShapeDtypeStruct((_C, _T, _H, _W), jnp.float32),
    ),
    mesh=plsc.VectorSubcoreMesh(core_axis_name="c", subcore_axis_name="s"),
    scratch_types=(
        [pltpu.VMEM((1, _HC, _W), jnp.float32)] * _NBUF
        + [pltpu.SemaphoreType.DMA] * (2 * _NBUF)
    ),
)
def _pack_pathway(x, slow, fast, *scratch):
    bufs = scratch[:_NBUF]
    sems_in = scratch[_NBUF:2 * _NBUF]
    sems_out = scratch[2 * _NBUF:]
    _body(x, slow, fast, bufs, sems_in, sems_out)


def kernel(frames):
    return _pack_pathway(frames)


# speedup vs baseline: 1.1471x; 1.0176x over previous
import functools

import jax
import jax.numpy as jnp
from jax import lax
from jax.experimental import pallas as pl
from jax.experimental.pallas import tpu as pltpu
from jax.experimental.pallas import tpu_sc as plsc

_C, _T, _H, _W = 3, 64, 256, 256
_TS = _T // 4

_NROWS = _C * _T
_NC, _NS = 2, 16
_NW = _NC * _NS
_FPW = _NROWS // _NW
_HC = 128
_CPR = _H // _HC
_NBUF = 3
_NCHUNK = _FPW * _CPR


def _body(x, slow, fast, bufs, sems_in, sems_out):
    cid = lax.axis_index("c")
    sid = lax.axis_index("s")
    w = sid * _NC + cid

    ins = []
    out_fast = []
    out_slow = []
    slow_flags = []
    for k in range(_NCHUNK):
        r = w * _FPW + (k // _CPR)
        c = r // _T
        t = r % _T
        h0 = (k % _CPR) * _HC
        b = k % _NBUF
        src = x.at[c, pl.ds(t, 1), pl.ds(h0, _HC)]
        ins.append(pltpu.make_async_copy(src, bufs[b], sems_in[b]))
        out_fast.append(pltpu.make_async_copy(
            bufs[b], fast.at[c, pl.ds(t, 1), pl.ds(h0, _HC)], sems_out[b]))
        kk = (5 * t + 10) // 21
        slow_flags.append((21 * kk) // 5 == t)
        out_slow.append(pltpu.make_async_copy(
            bufs[b], slow.at[c, pl.ds(kk, 1), pl.ds(h0, _HC)], sems_out[b]))

    def drain(j):
        out_fast[j].wait()

        @pl.when(slow_flags[j])
        def _():
            out_slow[j].wait()

    ins[0].start()
    ins[1].start()
    for k in range(_NCHUNK):
        ins[k].wait()
        out_fast[k].start()

        @pl.when(slow_flags[k])
        def _(k=k):
            out_slow[k].start()

        if k + 2 < _NCHUNK:
            if k - 1 >= 0:
                drain(k - 1)
            ins[k + 2].start()
    for j in range(_NCHUNK - _NBUF, _NCHUNK):
        drain(j)


@functools.partial(
    pl.kernel,
    out_type=(
        jax.ShapeDtypeStruct((_C, _TS, _H, _W), jnp.float32),
        jax.ShapeDtypeStruct((_C, _T, _H, _W), jnp.float32),
    ),
    mesh=plsc.VectorSubcoreMesh(core_axis_name="c", subcore_axis_name="s"),
    scratch_types=[
        pltpu.VMEM((1, _HC, _W), jnp.float32),
        pltpu.VMEM((1, _HC, _W), jnp.float32),
        pltpu.VMEM((1, _HC, _W), jnp.float32),
        pltpu.SemaphoreType.DMA,
        pltpu.SemaphoreType.DMA,
        pltpu.SemaphoreType.DMA,
        pltpu.SemaphoreType.DMA,
        pltpu.SemaphoreType.DMA,
        pltpu.SemaphoreType.DMA,
    ],
)
def _pack_pathway(x, slow, fast, b0, b1, b2, si0, si1, si2, so0, so1, so2):
    _body(x, slow, fast, (b0, b1, b2), (si0, si1, si2), (so0, so1, so2))


def kernel(frames):
    return _pack_pathway(frames)
